# Initial kernel scaffold; baseline (speedup 1.0000x reference)
import jax
import jax.numpy as jnp
from jax.experimental import pallas as pl


def kernel(x, token_table, pos_table, ln_gamma, ln_beta):
    raise NotImplementedError("write your pallas kernel here")


# speedup vs baseline: 2.3539x; 2.3539x over previous
import jax
import jax.numpy as jnp
from jax import lax
from jax.experimental import pallas as pl
from jax.experimental.pallas import tpu as pltpu
from jax.experimental.pallas import tpu_sc as plsc

B = 1024
L = 200
LP = 208
HIDDEN = 128
NC = 2
NS = 16
NW = NC * NS
SEQ_PER_W = B // NW
NV = HIDDEN // 16
NG = LP // 16
EPS = 1e-5 * HIDDEN


def _rsqrt(v):
    i = lax.bitcast_convert_type(v, jnp.int32)
    i = jnp.int32(0x5F3759DF) - (i >> 1)
    y = lax.bitcast_convert_type(i, jnp.float32)
    for _ in range(4):
        y = y * (1.5 - 0.5 * v * y * y)
    return y


def _body(x_hbm, tok_hbm, pos_hbm, g_hbm, b_hbm, out_hbm,
          idx_v, row_v, pos_v, xT_v, g_v, b_v, mean_a, rstd_a, sem):
    wid = lax.axis_index("s") * NC + lax.axis_index("c")

    pltpu.sync_copy(pos_hbm, pos_v)
    pltpu.sync_copy(g_hbm, g_v)
    pltpu.sync_copy(b_hbm, b_v)
    gs = [g_v[pl.ds(k * 16, 16)] for k in range(NV)]
    bs = [b_v[pl.ds(k * 16, 16)] for k in range(NV)]
    lanes = lax.iota(jnp.int32, 16)
    pre = [(k * 16 + lanes) * LP for k in range(NV)]

    def add_t_body(r, carry):
        for k in range(NV):
            x = row_v[r, pl.ds(k * 16, 16)] + pos_v[r, pl.ds(k * 16, 16)]
            row_v[r, pl.ds(k * 16, 16)] = x
            plsc.store_scatter(xT_v, [pre[k] + r], x)
        return carry

    def grp_body(g, carry):
        r0 = pl.multiple_of(g * 16, 16)
        acc = jnp.zeros((16,), jnp.float32)
        asq = jnp.zeros((16,), jnp.float32)
        for h in range(HIDDEN):
            x = xT_v[pl.ds(h * LP + r0, 16)]
            acc = acc + x
            asq = asq + x * x
        mean = acc * (1.0 / HIDDEN)
        var = asq * (1.0 / HIDDEN) - mean * mean + EPS
        mean_a[pl.ds(r0, 16)] = mean
        rstd_a[pl.ds(r0, 16)] = _rsqrt(var)
        return carry

    def row_body(r, carry):
        rr = jnp.full((16,), r, jnp.int32)
        mb = plsc.load_gather(mean_a, [rr])
        rb = plsc.load_gather(rstd_a, [rr])
        for k in range(NV):
            xk = row_v[r, pl.ds(k * 16, 16)]
            row_v[r, pl.ds(k * 16, 16)] = (xk - mb) * rb * gs[k] + bs[k]
        return carry

    def seq_body(s, carry):
        seq = wid * SEQ_PER_W + s
        base = pl.multiple_of(seq * L, 8)
        pltpu.sync_copy(x_hbm.at[pl.ds(base, L)], idx_v)
        pltpu.async_copy(
            tok_hbm.at[idx_v.at[pl.ds(0, 104)]],
            row_v.at[pl.ds(0, 104)], sem).wait()
        pltpu.async_copy(
            tok_hbm.at[idx_v.at[pl.ds(104, 96)]],
            row_v.at[pl.ds(104, 96)], sem).wait()
        lax.fori_loop(0, L, add_t_body, 0, unroll=False)
        lax.fori_loop(0, NG, grp_body, 0, unroll=False)
        lax.fori_loop(0, L, row_body, 0, unroll=False)
        pltpu.sync_copy(row_v.at[pl.ds(0, L)], out_hbm.at[pl.ds(base, L)])
        return carry

    lax.fori_loop(0, SEQ_PER_W, seq_body, 0, unroll=False)


@jax.jit
def _run(x_flat, token_table, pos_pad, ln_gamma, ln_beta):
    mesh = plsc.VectorSubcoreMesh(
        core_axis_name="c", subcore_axis_name="s",
        num_cores=NC, num_subcores=NS)
    return pl.kernel(
        _body,
        out_type=jax.ShapeDtypeStruct((B * L, HIDDEN), jnp.float32),
        mesh=mesh,
        compiler_params=pltpu.CompilerParams(needs_layout_passes=False),
        scratch_types=[
            pltpu.VMEM((L,), jnp.int32),
            pltpu.VMEM((LP, HIDDEN), jnp.float32),
            pltpu.VMEM((LP, HIDDEN), jnp.float32),
            pltpu.VMEM((HIDDEN * LP,), jnp.float32),
            pltpu.VMEM((HIDDEN,), jnp.float32),
            pltpu.VMEM((HIDDEN,), jnp.float32),
            pltpu.VMEM((LP,), jnp.float32),
            pltpu.VMEM((LP,), jnp.float32),
            pltpu.SemaphoreType.DMA,
        ],
    )(x_flat, token_table, pos_pad, ln_gamma, ln_beta)


def kernel(x, token_table, pos_table, ln_gamma, ln_beta):
    x_flat = x.reshape(-1).astype(jnp.int32)
    pos_pad = jnp.zeros((LP, HIDDEN), jnp.float32).at[:L].set(pos_table[:L])
    out = _run(x_flat, token_table, pos_pad, ln_gamma, ln_beta)
    return out.reshape(B, L, HIDDEN)


# speedup vs baseline: 2.6961x; 1.1454x over previous
import jax
import jax.numpy as jnp
from jax import lax
from jax.experimental import pallas as pl
from jax.experimental.pallas import tpu as pltpu
from jax.experimental.pallas import tpu_sc as plsc

B = 1024
L = 200
LP = 208
HIDDEN = 128
NC = 2
NS = 16
NW = NC * NS
SEQ_PER_W = B // NW
NTOK_W = SEQ_PER_W * L
NV = HIDDEN // 16
NG = LP // 16
EPS = 1e-5 * HIDDEN


def _rsqrt(v):
    i = lax.bitcast_convert_type(v, jnp.int32)
    i = jnp.int32(0x5F3759DF) - (i >> 1)
    y = lax.bitcast_convert_type(i, jnp.float32)
    for _ in range(4):
        y = y * (1.5 - 0.5 * v * y * y)
    return y


def _body(x_hbm, tok_hbm, pos_hbm, g_hbm, b_hbm, out_hbm,
          idx_v, row0, row1, pos_v, xT_v, g_v, b_v, mean_a, rstd_a,
          sa0, sb0, sa1, sb1, so0, so1):
    wid = lax.axis_index("s") * NC + lax.axis_index("c")
    w0 = pl.multiple_of(wid * NTOK_W, 8)

    pltpu.sync_copy(x_hbm.at[pl.ds(w0, NTOK_W)], idx_v)
    pltpu.sync_copy(pos_hbm, pos_v)
    pltpu.sync_copy(g_hbm, g_v)
    pltpu.sync_copy(b_hbm, b_v)
    gs = [g_v[pl.ds(k * 16, 16)] for k in range(NV)]
    bs = [b_v[pl.ds(k * 16, 16)] for k in range(NV)]
    lanes = lax.iota(jnp.int32, 16)
    pre = [(k * 16 + lanes) * LP for k in range(NV)]

    rows = (row0, row1)
    sas = (sa0, sa1)
    sbs = (sb0, sb1)
    sos = (so0, so1)

    def gather_copies(i, p):
        soff = pl.multiple_of(i * L, 8)
        ca = (tok_hbm.at[idx_v.at[pl.ds(soff, 104)]],
              rows[p].at[pl.ds(0, 104)], sas[p])
        cb = (tok_hbm.at[idx_v.at[pl.ds(soff + 104, 96)]],
              rows[p].at[pl.ds(104, 96)], sbs[p])
        return ca, cb

    def start_gather(i, p):
        ca, cb = gather_copies(i, p)
        pltpu.async_copy(*ca)
        pltpu.async_copy(*cb)

    def wait_gather(i, p):
        ca, cb = gather_copies(i, p)
        pltpu.make_async_copy(*ca).wait()
        pltpu.make_async_copy(*cb).wait()

    def start_out(i, p):
        base = pl.multiple_of((wid * SEQ_PER_W + i) * L, 8)
        pltpu.async_copy(
            rows[p].at[pl.ds(0, L)], out_hbm.at[pl.ds(base, L)], sos[p])

    def wait_out(p):
        pltpu.make_async_copy(
            rows[p].at[pl.ds(0, L)], out_hbm.at[pl.ds(0, L)], sos[p]).wait()

    def compute(p):
        row_v = rows[p]

        def add_t_body(r, carry):
            for k in range(NV):
                x = row_v[r, pl.ds(k * 16, 16)] + pos_v[r, pl.ds(k * 16, 16)]
                row_v[r, pl.ds(k * 16, 16)] = x
                plsc.store_scatter(xT_v, [pre[k] + r], x)
            return carry

        def grp_body(g, carry):
            r0 = pl.multiple_of(g * 16, 16)
            acc = jnp.zeros((16,), jnp.float32)
            asq = jnp.zeros((16,), jnp.float32)
            for h in range(HIDDEN):
                xh = xT_v[pl.ds(h * LP + r0, 16)]
                acc = acc + xh
                asq = asq + xh * xh
            mean = acc * (1.0 / HIDDEN)
            var = asq * (1.0 / HIDDEN) - mean * mean + EPS
            mean_a[pl.ds(r0, 16)] = mean
            rstd_a[pl.ds(r0, 16)] = _rsqrt(var)
            return carry

        def row_body(r, carry):
            rr = jnp.full((16,), r, jnp.int32)
            mb = plsc.load_gather(mean_a, [rr])
            rb = plsc.load_gather(rstd_a, [rr])
            for k in range(NV):
                xk = row_v[r, pl.ds(k * 16, 16)]
                row_v[r, pl.ds(k * 16, 16)] = (xk - mb) * rb * gs[k] + bs[k]
            return carry

        lax.fori_loop(0, L, add_t_body, 0, unroll=False)
        lax.fori_loop(0, NG, grp_body, 0, unroll=False)
        lax.fori_loop(0, L, row_body, 0, unroll=False)

    start_gather(0, 0)

    def pair_body(s2, carry):
        for p in (0, 1):
            i = 2 * s2 + p
            q = 1 - p
            wait_gather(i, p)
            inext = jnp.minimum(i + 1, SEQ_PER_W - 1)
            start_gather(inext, q)
            compute(p)
            start_out(i, p)
            wait_out(p)
        return carry

    lax.fori_loop(0, SEQ_PER_W // 2, pair_body, 0, unroll=False)
    wait_gather(SEQ_PER_W - 1, 0)


@jax.jit
def _run(x_flat, token_table, pos_pad, ln_gamma, ln_beta):
    mesh = plsc.VectorSubcoreMesh(
        core_axis_name="c", subcore_axis_name="s",
        num_cores=NC, num_subcores=NS)
    return pl.kernel(
        _body,
        out_type=jax.ShapeDtypeStruct((B * L, HIDDEN), jnp.float32),
        mesh=mesh,
        compiler_params=pltpu.CompilerParams(needs_layout_passes=False),
        scratch_types=[
            pltpu.VMEM((NTOK_W,), jnp.int32),
            pltpu.VMEM((LP, HIDDEN), jnp.float32),
            pltpu.VMEM((LP, HIDDEN), jnp.float32),
            pltpu.VMEM((LP, HIDDEN), jnp.float32),
            pltpu.VMEM((HIDDEN * LP,), jnp.float32),
            pltpu.VMEM((HIDDEN,), jnp.float32),
            pltpu.VMEM((HIDDEN,), jnp.float32),
            pltpu.VMEM((LP,), jnp.float32),
            pltpu.VMEM((LP,), jnp.float32),
            pltpu.SemaphoreType.DMA,
            pltpu.SemaphoreType.DMA,
            pltpu.SemaphoreType.DMA,
            pltpu.SemaphoreType.DMA,
            pltpu.SemaphoreType.DMA,
            pltpu.SemaphoreType.DMA,
        ],
    )(x_flat, token_table, pos_pad, ln_gamma, ln_beta)


def kernel(x, token_table, pos_table, ln_gamma, ln_beta):
    x_flat = x.reshape(-1).astype(jnp.int32)
    pos_pad = jnp.zeros((LP, HIDDEN), jnp.float32).at[:L].set(pos_table[:L])
    out = _run(x_flat, token_table, pos_pad, ln_gamma, ln_beta)
    return out.reshape(B, L, HIDDEN)


# speedup vs baseline: 4.1138x; 1.5259x over previous
import jax
import jax.numpy as jnp
from jax import lax
from jax.experimental import pallas as pl
from jax.experimental.pallas import tpu as pltpu
from jax.experimental.pallas import tpu_sc as plsc

B = 1024
L = 200
LP = 208
HIDDEN = 128
NC = 2
NS = 16
NW = NC * NS
SEQ_PER_W = B // NW
NTOK_W = SEQ_PER_W * L
NV = HIDDEN // 16
NG = LP // 16
EPS = 1e-5 * HIDDEN


_PERM_DNUMS = lax.GatherDimensionNumbers(
    offset_dims=(), collapsed_slice_dims=(0,), start_index_map=(0,))


def _permute(v, pm):
    return lax.gather(v, pm[:, None], _PERM_DNUMS, slice_sizes=(1,),
                      mode=lax.GatherScatterMode.PROMISE_IN_BOUNDS)


def _rsqrt(v):
    i = lax.bitcast_convert_type(v, jnp.int32)
    i = jnp.int32(0x5F3759DF) - (i >> 1)
    y = lax.bitcast_convert_type(i, jnp.float32)
    for _ in range(4):
        y = y * (1.5 - 0.5 * v * y * y)
    return y


def _body(x_hbm, tok_hbm, pos_hbm, g_hbm, b_hbm, out_hbm,
          idx_v, row0, row1, pos_v, xT_v, g_v, b_v, mean_a, rstd_a,
          sa0, sb0, sa1, sb1, so0, so1):
    wid = lax.axis_index("s") * NC + lax.axis_index("c")
    w0 = pl.multiple_of(wid * NTOK_W, 8)

    pltpu.sync_copy(x_hbm.at[pl.ds(w0, NTOK_W)], idx_v)
    pltpu.sync_copy(pos_hbm, pos_v)
    pltpu.sync_copy(g_hbm, g_v)
    pltpu.sync_copy(b_hbm, b_v)
    gs = [g_v[pl.ds(k * 16, 16)] for k in range(NV)]
    bs = [b_v[pl.ds(k * 16, 16)] for k in range(NV)]
    lanes = lax.iota(jnp.int32, 16)
    pre = [(k * 16 + lanes) * LP for k in range(NV)]

    rows = (row0, row1)
    sas = (sa0, sa1)
    sbs = (sb0, sb1)
    sos = (so0, so1)

    def gather_copies(i, p):
        soff = pl.multiple_of(i * L, 8)
        ca = (tok_hbm.at[idx_v.at[pl.ds(soff, 104)]],
              rows[p].at[pl.ds(0, 104)], sas[p])
        cb = (tok_hbm.at[idx_v.at[pl.ds(soff + 104, 96)]],
              rows[p].at[pl.ds(104, 96)], sbs[p])
        return ca, cb

    def start_gather(i, p):
        ca, cb = gather_copies(i, p)
        pltpu.async_copy(*ca)
        pltpu.async_copy(*cb)

    def wait_gather(i, p):
        ca, cb = gather_copies(i, p)
        pltpu.make_async_copy(*ca).wait()
        pltpu.make_async_copy(*cb).wait()

    def start_out(i, p):
        base = pl.multiple_of((wid * SEQ_PER_W + i) * L, 8)
        pltpu.async_copy(
            rows[p].at[pl.ds(0, L)], out_hbm.at[pl.ds(base, L)], sos[p])

    def wait_out(p):
        pltpu.make_async_copy(
            rows[p].at[pl.ds(0, L)], out_hbm.at[pl.ds(0, L)], sos[p]).wait()

    perms = [lanes ^ d for d in (8, 4, 2, 1)]

    def compute(p):
        row_v = rows[p]

        def row_body(r, carry):
            xs = []
            for k in range(NV):
                xs.append(
                    row_v[r, pl.ds(k * 16, 16)] + pos_v[r, pl.ds(k * 16, 16)])
            acc = xs[0]
            ssq = xs[0] * xs[0]
            for k in range(1, NV):
                acc = acc + xs[k]
                ssq = ssq + xs[k] * xs[k]
            for pm in perms:
                acc = acc + _permute(acc, pm)
                ssq = ssq + _permute(ssq, pm)
            mean = acc * (1.0 / HIDDEN)
            var = ssq * (1.0 / HIDDEN) - mean * mean + EPS
            rstd = _rsqrt(var)
            for k in range(NV):
                row_v[r, pl.ds(k * 16, 16)] = \
                    (xs[k] - mean) * rstd * gs[k] + bs[k]
            return carry

        lax.fori_loop(0, L, row_body, 0, unroll=False)

    start_gather(0, 0)

    def pair_body(s2, carry):
        for p in (0, 1):
            i = 2 * s2 + p
            q = 1 - p
            wait_gather(i, p)
            inext = jnp.minimum(i + 1, SEQ_PER_W - 1)
            start_gather(inext, q)
            compute(p)
            start_out(i, p)
            wait_out(p)
        return carry

    lax.fori_loop(0, SEQ_PER_W // 2, pair_body, 0, unroll=False)
    wait_gather(SEQ_PER_W - 1, 0)


@jax.jit
def _run(x_flat, token_table, pos_pad, ln_gamma, ln_beta):
    mesh = plsc.VectorSubcoreMesh(
        core_axis_name="c", subcore_axis_name="s",
        num_cores=NC, num_subcores=NS)
    return pl.kernel(
        _body,
        out_type=jax.ShapeDtypeStruct((B * L, HIDDEN), jnp.float32),
        mesh=mesh,
        compiler_params=pltpu.CompilerParams(needs_layout_passes=False),
        scratch_types=[
            pltpu.VMEM((NTOK_W,), jnp.int32),
            pltpu.VMEM((LP, HIDDEN), jnp.float32),
            pltpu.VMEM((LP, HIDDEN), jnp.float32),
            pltpu.VMEM((LP, HIDDEN), jnp.float32),
            pltpu.VMEM((HIDDEN * LP,), jnp.float32),
            pltpu.VMEM((HIDDEN,), jnp.float32),
            pltpu.VMEM((HIDDEN,), jnp.float32),
            pltpu.VMEM((LP,), jnp.float32),
            pltpu.VMEM((LP,), jnp.float32),
            pltpu.SemaphoreType.DMA,
            pltpu.SemaphoreType.DMA,
            pltpu.SemaphoreType.DMA,
            pltpu.SemaphoreType.DMA,
            pltpu.SemaphoreType.DMA,
            pltpu.SemaphoreType.DMA,
        ],
    )(x_flat, token_table, pos_pad, ln_gamma, ln_beta)


def kernel(x, token_table, pos_table, ln_gamma, ln_beta):
    x_flat = x.reshape(-1).astype(jnp.int32)
    pos_pad = jnp.zeros((LP, HIDDEN), jnp.float32).at[:L].set(pos_table[:L])
    out = _run(x_flat, token_table, pos_pad, ln_gamma, ln_beta)
    return out.reshape(B, L, HIDDEN)


# speedup vs baseline: 7.2768x; 1.7689x over previous
import jax
import jax.numpy as jnp
from jax import lax
from jax.experimental import pallas as pl
from jax.experimental.pallas import tpu as pltpu
from jax.experimental.pallas import tpu_sc as plsc

B = 1024
L = 200
LP = 208
HIDDEN = 128
NC = 2
NS = 16
NW = NC * NS
SEQ_PER_W = B // NW
NTOK_W = SEQ_PER_W * L
NV = HIDDEN // 16
NG = LP // 16
EPS = 1e-5 * HIDDEN


_PERM_DNUMS = lax.GatherDimensionNumbers(
    offset_dims=(), collapsed_slice_dims=(0,), start_index_map=(0,))


def _permute(v, pm):
    return lax.gather(v, pm[:, None], _PERM_DNUMS, slice_sizes=(1,),
                      mode=lax.GatherScatterMode.PROMISE_IN_BOUNDS)


def _rsqrt(v):
    i = lax.bitcast_convert_type(v, jnp.int32)
    i = jnp.int32(0x5F3759DF) - (i >> 1)
    y = lax.bitcast_convert_type(i, jnp.float32)
    for _ in range(4):
        y = y * (1.5 - 0.5 * v * y * y)
    return y


def _body(x_hbm, tok_hbm, pos_hbm, g_hbm, b_hbm, out_hbm,
          idx_v, row0, row1, pos_v, xT_v, g_v, b_v, mean_a, rstd_a,
          sa0, sb0, sa1, sb1, so0, so1):
    wid = lax.axis_index("s") * NC + lax.axis_index("c")
    w0 = pl.multiple_of(wid * NTOK_W, 8)

    pltpu.sync_copy(x_hbm.at[pl.ds(w0, NTOK_W)], idx_v)
    pltpu.sync_copy(pos_hbm, pos_v)
    pltpu.sync_copy(g_hbm, g_v)
    pltpu.sync_copy(b_hbm, b_v)
    gs = [g_v[pl.ds(k * 16, 16)] for k in range(NV)]
    bs = [b_v[pl.ds(k * 16, 16)] for k in range(NV)]
    lanes = lax.iota(jnp.int32, 16)
    pre = [(k * 16 + lanes) * LP for k in range(NV)]

    rows = (row0, row1)
    sas = (sa0, sa1)
    sbs = (sb0, sb1)
    sos = (so0, so1)

    def gather_copies(i, p):
        soff = pl.multiple_of(i * L, 8)
        ca = (tok_hbm.at[idx_v.at[pl.ds(soff, 104)]],
              rows[p].at[pl.ds(0, 104)], sas[p])
        cb = (tok_hbm.at[idx_v.at[pl.ds(soff + 104, 96)]],
              rows[p].at[pl.ds(104, 96)], sbs[p])
        return ca, cb

    def start_gather(i, p):
        ca, cb = gather_copies(i, p)
        pltpu.async_copy(*ca)
        pltpu.async_copy(*cb)

    def wait_gather(i, p):
        ca, cb = gather_copies(i, p)
        pltpu.make_async_copy(*ca).wait()
        pltpu.make_async_copy(*cb).wait()

    def start_out(i, p):
        base = pl.multiple_of((wid * SEQ_PER_W + i) * L, 8)
        pltpu.async_copy(
            rows[p].at[pl.ds(0, L)], out_hbm.at[pl.ds(base, L)], sos[p])

    def wait_out(p):
        pltpu.make_async_copy(
            rows[p].at[pl.ds(0, L)], out_hbm.at[pl.ds(0, L)], sos[p]).wait()

    perms = [lanes ^ d for d in (8, 4, 2, 1)]

    def compute(p):
        row_v = rows[p]

        @plsc.parallel_loop(0, L, unroll=2)
        def _(r):
            xs = []
            for k in range(NV):
                xs.append(
                    row_v[r, pl.ds(k * 16, 16)] + pos_v[r, pl.ds(k * 16, 16)])
            acc = xs[0]
            ssq = xs[0] * xs[0]
            for k in range(1, NV):
                acc = acc + xs[k]
                ssq = ssq + xs[k] * xs[k]
            for pm in perms:
                acc = acc + _permute(acc, pm)
                ssq = ssq + _permute(ssq, pm)
            mean = acc * (1.0 / HIDDEN)
            var = ssq * (1.0 / HIDDEN) - mean * mean + EPS
            rstd = _rsqrt(var)
            for k in range(NV):
                row_v[r, pl.ds(k * 16, 16)] = \
                    (xs[k] - mean) * rstd * gs[k] + bs[k]

    start_gather(0, 0)

    def pair_body(s2, carry):
        for p in (0, 1):
            i = 2 * s2 + p
            q = 1 - p
            wait_gather(i, p)
            inext = jnp.minimum(i + 1, SEQ_PER_W - 1)
            start_gather(inext, q)
            compute(p)
            start_out(i, p)
            wait_out(p)
        return carry

    lax.fori_loop(0, SEQ_PER_W // 2, pair_body, 0, unroll=False)
    wait_gather(SEQ_PER_W - 1, 0)


@jax.jit
def _run(x_flat, token_table, pos_pad, ln_gamma, ln_beta):
    mesh = plsc.VectorSubcoreMesh(
        core_axis_name="c", subcore_axis_name="s",
        num_cores=NC, num_subcores=NS)
    return pl.kernel(
        _body,
        out_type=jax.ShapeDtypeStruct((B * L, HIDDEN), jnp.float32),
        mesh=mesh,
        compiler_params=pltpu.CompilerParams(needs_layout_passes=False),
        scratch_types=[
            pltpu.VMEM((NTOK_W,), jnp.int32),
            pltpu.VMEM((LP, HIDDEN), jnp.float32),
            pltpu.VMEM((LP, HIDDEN), jnp.float32),
            pltpu.VMEM((LP, HIDDEN), jnp.float32),
            pltpu.VMEM((HIDDEN * LP,), jnp.float32),
            pltpu.VMEM((HIDDEN,), jnp.float32),
            pltpu.VMEM((HIDDEN,), jnp.float32),
            pltpu.VMEM((LP,), jnp.float32),
            pltpu.VMEM((LP,), jnp.float32),
            pltpu.SemaphoreType.DMA,
            pltpu.SemaphoreType.DMA,
            pltpu.SemaphoreType.DMA,
            pltpu.SemaphoreType.DMA,
            pltpu.SemaphoreType.DMA,
            pltpu.SemaphoreType.DMA,
        ],
    )(x_flat, token_table, pos_pad, ln_gamma, ln_beta)


def kernel(x, token_table, pos_table, ln_gamma, ln_beta):
    x_flat = x.reshape(-1).astype(jnp.int32)
    pos_pad = jnp.zeros((LP, HIDDEN), jnp.float32).at[:L].set(pos_table[:L])
    out = _run(x_flat, token_table, pos_pad, ln_gamma, ln_beta)
    return out.reshape(B, L, HIDDEN)
